# baseline (speedup 1.0000x reference)
import jax
import jax.numpy as jnp
from jax.experimental import pallas as pl
from jax.experimental.pallas import tpu as pltpu

_NUM_CLASSES = 8192
_Z_DIM = 256
_HALF = _NUM_CLASSES // 2


def _copy_body(a_hbm, o_hbm, buf, in_sems, out_sems):
    lo = pl.ds(0, _HALF)
    hi = pl.ds(_HALF, _HALF)
    pltpu.make_async_copy(a_hbm.at[lo, :], buf.at[lo, :], in_sems.at[0]).start()
    pltpu.make_async_copy(a_hbm.at[lo, :], buf.at[lo, :], in_sems.at[0]).wait()
    pltpu.make_async_copy(buf.at[lo, :], o_hbm.at[lo, :], out_sems.at[0]).start()
    pltpu.make_async_copy(a_hbm.at[hi, :], buf.at[hi, :], in_sems.at[1]).start()
    pltpu.make_async_copy(a_hbm.at[hi, :], buf.at[hi, :], in_sems.at[1]).wait()
    pltpu.make_async_copy(buf.at[hi, :], o_hbm.at[hi, :], out_sems.at[1]).start()
    pltpu.make_async_copy(buf.at[lo, :], o_hbm.at[lo, :], out_sems.at[0]).wait()
    pltpu.make_async_copy(buf.at[hi, :], o_hbm.at[hi, :], out_sems.at[1]).wait()


def kernel(_, anchor):
    return pl.pallas_call(
        _copy_body,
        in_specs=[pl.BlockSpec(memory_space=pl.ANY)],
        out_specs=pl.BlockSpec(memory_space=pl.ANY),
        out_shape=jax.ShapeDtypeStruct((_NUM_CLASSES, _Z_DIM), jnp.float32),
        scratch_shapes=[
            pltpu.VMEM((_NUM_CLASSES, _Z_DIM), jnp.float32),
            pltpu.SemaphoreType.DMA((2,)),
            pltpu.SemaphoreType.DMA((2,)),
        ],
    )(anchor)
